# Initial kernel scaffold; baseline (speedup 1.0000x reference)
import jax
import jax.numpy as jnp
from jax.experimental import pallas as pl


def kernel(h, W, a, adj):
    raise NotImplementedError("write your pallas kernel here")


# speedup vs baseline: 1.0639x; 1.0639x over previous
import functools

import jax
import jax.numpy as jnp
from jax.experimental import pallas as pl
from jax.experimental.pallas import tpu as pltpu


def _project_kernel(h_ref, w_ref, a_ref, whb_ref, sq_ref, sk_ref):
    wh = jnp.dot(h_ref[...], w_ref[...], preferred_element_type=jnp.float32)
    sc = jnp.dot(wh, a_ref[...], preferred_element_type=jnp.float32)
    whb_ref[...] = wh.astype(jnp.bfloat16)
    sq_ref[...] = sc[:, 0:1]
    sk_ref[...] = jnp.transpose(sc[:, 1:2])


def _attend_kernel(whb_ref, sq_ref, sk_ref, adj_ref, out_ref, *, alpha):
    sk = sk_ref[...]
    sq = sq_ref[...]
    rm = sq + jnp.max(sk)
    m = jnp.maximum(rm, alpha * rm)

    x = sq + sk
    e = jnp.maximum(x, alpha * x)
    p = jnp.exp(e - m) * adj_ref[...]
    denom = jnp.sum(p, axis=1, keepdims=True)

    acc = jnp.dot(p.astype(jnp.bfloat16), whb_ref[...],
                  preferred_element_type=jnp.float32)
    out = acc * pl.reciprocal(denom, approx=False)
    out = jnp.where(out > 0, out, jnp.exp(out) - 1.0)
    out_ref[...] = out


def _row_tile(n, max_tile=512):
    if n <= max_tile:
        return n
    for t in (512, 256, 128):
        if n % t == 0:
            return t
    return n


def kernel(h, W, a, adj):
    alpha = 0.2
    N, f_in = h.shape
    f_out = W.shape[1]
    a_mat = jnp.transpose(a.reshape(2, f_out))

    tq = _row_tile(N)
    n_tiles = N // tq

    proj_cost = pl.CostEstimate(
        flops=2 * N * f_in * f_out + 4 * N * f_out,
        transcendentals=0,
        bytes_accessed=4 * (N * f_in + f_in * f_out + 2 * f_out + 2 * N)
        + 2 * N * f_out,
    )
    whb, s_q, s_k = pl.pallas_call(
        _project_kernel,
        out_shape=(
            jax.ShapeDtypeStruct((N, f_out), jnp.bfloat16),
            jax.ShapeDtypeStruct((N, 1), jnp.float32),
            jax.ShapeDtypeStruct((1, N), jnp.float32),
        ),
        grid=(n_tiles,),
        in_specs=[
            pl.BlockSpec((tq, f_in), lambda i: (i, 0)),
            pl.BlockSpec((f_in, f_out), lambda i: (0, 0)),
            pl.BlockSpec((f_out, 2), lambda i: (0, 0)),
        ],
        out_specs=(
            pl.BlockSpec((tq, f_out), lambda i: (i, 0)),
            pl.BlockSpec((tq, 1), lambda i: (i, 0)),
            pl.BlockSpec((1, tq), lambda i: (0, i)),
        ),
        compiler_params=pltpu.CompilerParams(dimension_semantics=("parallel",)),
        cost_estimate=proj_cost,
    )(h, W, a_mat)

    attend_cost = pl.CostEstimate(
        flops=2 * N * N * f_out + 8 * N * N,
        transcendentals=N * N + N * f_out,
        bytes_accessed=4 * (N * N + N * f_out + 3 * N) + 2 * N * f_out,
    )
    attend = functools.partial(_attend_kernel, alpha=alpha)
    out = pl.pallas_call(
        attend,
        out_shape=jax.ShapeDtypeStruct((N, f_out), jnp.float32),
        grid=(n_tiles,),
        in_specs=[
            pl.BlockSpec((N, f_out), lambda i: (0, 0)),
            pl.BlockSpec((tq, 1), lambda i: (i, 0)),
            pl.BlockSpec((1, N), lambda i: (0, 0)),
            pl.BlockSpec((tq, N), lambda i: (i, 0)),
        ],
        out_specs=pl.BlockSpec((tq, f_out), lambda i: (i, 0)),
        compiler_params=pltpu.CompilerParams(dimension_semantics=("parallel",)),
        cost_estimate=attend_cost,
    )(whb, s_q, s_k, adj)
    return out


# speedup vs baseline: 1.2970x; 1.2190x over previous
import functools

import jax
import jax.numpy as jnp
from jax.experimental import pallas as pl
from jax.experimental.pallas import tpu as pltpu

_LOG2E = 1.4426950408889634


def _gat_kernel(h_ref, w_ref, wa_ref, adj_ref, out_ref,
                whb_ref, sq_ref, sk_ref, *, alpha, tq, n_tiles):
    i = pl.program_id(0)

    @pl.when(i == 0)
    def _project():
        for c in range(n_tiles):
            hc = h_ref[c * tq:(c + 1) * tq, :]
            wh = jnp.dot(hc, w_ref[...], preferred_element_type=jnp.float32)
            whb_ref[c * tq:(c + 1) * tq, :] = wh.astype(jnp.bfloat16)
            sc = jnp.dot(hc, wa_ref[...], preferred_element_type=jnp.float32)
            sq_ref[c * tq:(c + 1) * tq, :] = sc[:, 0:1]
            sk_ref[0:1, c * tq:(c + 1) * tq] = jnp.transpose(sc[:, 1:2])

    sk = sk_ref[...]
    sq = sq_ref[pl.ds(i * tq, tq), :]
    rm = sq + jnp.max(sk)
    m = jnp.maximum(rm, alpha * rm)

    a1 = (sq - m) * _LOG2E
    a2 = (alpha * sq - m) * _LOG2E
    b1 = sk * _LOG2E
    b2 = sk * (alpha * _LOG2E)
    t = jnp.maximum(a1 + b1, a2 + b2)
    p = jnp.exp2(t) * adj_ref[...]
    denom = jnp.sum(p, axis=1, keepdims=True)

    acc = jnp.dot(p.astype(jnp.bfloat16), whb_ref[...],
                  preferred_element_type=jnp.float32)
    out = acc * pl.reciprocal(denom, approx=False)
    out = jnp.where(out > 0, out, jnp.exp(out) - 1.0)
    out_ref[...] = out


def _row_tile(n, max_tile=512):
    if n <= max_tile:
        return n
    for t in (512, 256, 128):
        if n % t == 0:
            return t
    return n


def kernel(h, W, a, adj):
    alpha = 0.2
    N, f_in = h.shape
    f_out = W.shape[1]
    a_mat = jnp.transpose(a.reshape(2, f_out))
    wa = jnp.dot(W, a_mat)

    tq = _row_tile(N)
    n_tiles = N // tq

    cost = pl.CostEstimate(
        flops=2 * N * f_in * f_out + 4 * N * f_in + 2 * N * N * f_out
        + 8 * N * N,
        transcendentals=N * N + N * f_out,
        bytes_accessed=4 * (N * N + N * f_in + N * f_out + f_in * f_out),
    )
    body = functools.partial(_gat_kernel, alpha=alpha, tq=tq, n_tiles=n_tiles)
    out = pl.pallas_call(
        body,
        out_shape=jax.ShapeDtypeStruct((N, f_out), jnp.float32),
        grid=(n_tiles,),
        in_specs=[
            pl.BlockSpec((N, f_in), lambda i: (0, 0)),
            pl.BlockSpec((f_in, f_out), lambda i: (0, 0)),
            pl.BlockSpec((f_in, 2), lambda i: (0, 0)),
            pl.BlockSpec((tq, N), lambda i: (i, 0)),
        ],
        out_specs=pl.BlockSpec((tq, f_out), lambda i: (i, 0)),
        scratch_shapes=[
            pltpu.VMEM((N, f_out), jnp.bfloat16),
            pltpu.VMEM((N, 1), jnp.float32),
            pltpu.VMEM((1, N), jnp.float32),
        ],
        compiler_params=pltpu.CompilerParams(
            dimension_semantics=("arbitrary",)),
        cost_estimate=cost,
    )(h, W, wa, adj)
    return out
